# baseline (speedup 1.0000x reference)
import jax
import jax.numpy as jnp
from jax.experimental import pallas as pl

_HB = 8


def _fused_heads_body(x_ref, wc_ref, bc_ref, wr_ref, br_ref, wd_ref, bd_ref,
                      cls_ref, reg_ref, dir_ref):
    c, hb, w = x_ref.shape[1], x_ref.shape[2], x_ref.shape[3]
    xb = x_ref[0].reshape(c, hb * w)
    cls_ref[0] = (jnp.dot(wc_ref[...], xb, preferred_element_type=jnp.float32)
                  + bc_ref[...]).reshape(cls_ref.shape[1], hb, w)
    reg_ref[0] = (jnp.dot(wr_ref[...], xb, preferred_element_type=jnp.float32)
                  + br_ref[...]).reshape(reg_ref.shape[1], hb, w)
    dir_ref[0] = (jnp.dot(wd_ref[...], xb, preferred_element_type=jnp.float32)
                  + bd_ref[...]).reshape(dir_ref.shape[1], hb, w)


@jax.jit
def kernel(x, W_cls, b_cls, W_reg, b_reg, W_dir, b_dir):
    B, C, H, W = x.shape

    def _wspec(o):
        return pl.BlockSpec((o, C), lambda b, j: (0, 0))

    def _bspec(o):
        return pl.BlockSpec((o, 1), lambda b, j: (0, 0))

    def _ospec(o):
        return pl.BlockSpec((1, o, _HB, W), lambda b, j: (b, 0, j, 0))

    o_cls, o_reg, o_dir = W_cls.shape[0], W_reg.shape[0], W_dir.shape[0]

    cls_o, reg_o, dir_o = pl.pallas_call(
        _fused_heads_body,
        grid=(B, H // _HB),
        in_specs=[
            pl.BlockSpec((1, C, _HB, W), lambda b, j: (b, 0, j, 0)),
            _wspec(o_cls), _bspec(o_cls),
            _wspec(o_reg), _bspec(o_reg),
            _wspec(o_dir), _bspec(o_dir),
        ],
        out_specs=(_ospec(o_cls), _ospec(o_reg), _ospec(o_dir)),
        out_shape=(
            jax.ShapeDtypeStruct((B, o_cls, H, W), jnp.float32),
            jax.ShapeDtypeStruct((B, o_reg, H, W), jnp.float32),
            jax.ShapeDtypeStruct((B, o_dir, H, W), jnp.float32),
        ),
    )(x,
      W_cls, b_cls.reshape(o_cls, 1),
      W_reg, b_reg.reshape(o_reg, 1),
      W_dir, b_dir.reshape(o_dir, 1))

    return (cls_o, reg_o, dir_o)


# speedup vs baseline: 1.1393x; 1.1393x over previous
import jax
import jax.numpy as jnp
from jax.experimental import pallas as pl

_CB = 64

_DN = (((0,), (0,)), ((), ()))


def _fused_heads_body(x_ref, wc_ref, bc_ref, wr_ref, br_ref, wd_ref, bd_ref,
                      cls_ref, reg_ref, dir_ref):
    j = pl.program_id(1)
    xb = x_ref[0]
    acc_c = jax.lax.dot_general(wc_ref[...], xb, _DN,
                                preferred_element_type=jnp.float32)
    acc_r = jax.lax.dot_general(wr_ref[...], xb, _DN,
                                preferred_element_type=jnp.float32)
    acc_d = jax.lax.dot_general(wd_ref[...], xb, _DN,
                                preferred_element_type=jnp.float32)

    @pl.when(j == 0)
    def _init():
        cls_ref[0] = acc_c + bc_ref[...]
        reg_ref[0] = acc_r + br_ref[...]
        dir_ref[0] = acc_d + bd_ref[...]

    @pl.when(j != 0)
    def _accum():
        cls_ref[0] += acc_c
        reg_ref[0] += acc_r
        dir_ref[0] += acc_d


@jax.jit
def kernel(x, W_cls, b_cls, W_reg, b_reg, W_dir, b_dir):
    B, C, H, W = x.shape
    n = H * W
    xf = x.reshape(B, C, n)

    def _wspec(o):
        return pl.BlockSpec((_CB, o), lambda b, j: (j, 0))

    def _bspec(o):
        return pl.BlockSpec((o, 1), lambda b, j: (0, 0))

    def _ospec(o):
        return pl.BlockSpec((1, o, n), lambda b, j: (b, 0, 0))

    o_cls, o_reg, o_dir = W_cls.shape[0], W_reg.shape[0], W_dir.shape[0]

    cls_f, reg_f, dir_f = pl.pallas_call(
        _fused_heads_body,
        grid=(B, C // _CB),
        in_specs=[
            pl.BlockSpec((1, _CB, n), lambda b, j: (b, j, 0)),
            _wspec(o_cls), _bspec(o_cls),
            _wspec(o_reg), _bspec(o_reg),
            _wspec(o_dir), _bspec(o_dir),
        ],
        out_specs=(_ospec(o_cls), _ospec(o_reg), _ospec(o_dir)),
        out_shape=(
            jax.ShapeDtypeStruct((B, o_cls, n), jnp.float32),
            jax.ShapeDtypeStruct((B, o_reg, n), jnp.float32),
            jax.ShapeDtypeStruct((B, o_dir, n), jnp.float32),
        ),
    )(xf,
      W_cls.T, b_cls.reshape(o_cls, 1),
      W_reg.T, b_reg.reshape(o_reg, 1),
      W_dir.T, b_dir.reshape(o_dir, 1))

    return (cls_f.reshape(B, o_cls, H, W),
            reg_f.reshape(B, o_reg, H, W),
            dir_f.reshape(B, o_dir, H, W))
